# Initial kernel scaffold; baseline (speedup 1.0000x reference)
import jax
import jax.numpy as jnp
from jax.experimental import pallas as pl


def kernel(x, edge_index, batch, W, a_src, a_dst, bias, W_lin, b_lin):
    raise NotImplementedError("write your pallas kernel here")


# speedup vs baseline: 20.7379x; 20.7379x over previous
import functools

import jax
import jax.numpy as jnp
from jax import lax
from jax.experimental import pallas as pl
from jax.experimental.pallas import tpu as pltpu
from jax.experimental.pallas import tpu_sc as plsc

NN = 10000
DD = 128
GG = 64
NOUT = 10
NP = 10240
EE = 320000
ET = EE + NN
NW = 32
CK = 128
CPT = 10368
ETP = NW * CPT
VPT = CPT // 16
NG = CPT // CK
RPT = NP // 16

_mesh = plsc.VectorSubcoreMesh(core_axis_name="c", subcore_axis_name="s")


def _prep_body(x_ref, w_ref, as_ref, ad_ref, h_ref, s_ref, d_ref):
    h = jnp.dot(x_ref[...], w_ref[...], preferred_element_type=jnp.float32)
    h_ref[...] = h
    s_ref[...] = jnp.sum(h * as_ref[...], axis=-1, keepdims=True)
    d_ref[...] = jnp.sum(h * ad_ref[...], axis=-1, keepdims=True)


def _prep(x_pad, W, a_src, a_dst):
    nb = NP // 1280
    return pl.pallas_call(
        _prep_body,
        grid=(nb,),
        in_specs=[
            pl.BlockSpec((1280, DD), lambda i: (i, 0)),
            pl.BlockSpec((DD, DD), lambda i: (0, 0)),
            pl.BlockSpec((1, DD), lambda i: (0, 0)),
            pl.BlockSpec((1, DD), lambda i: (0, 0)),
        ],
        out_specs=[
            pl.BlockSpec((1280, DD), lambda i: (i, 0)),
            pl.BlockSpec((1280, 1), lambda i: (i, 0)),
            pl.BlockSpec((1280, 1), lambda i: (i, 0)),
        ],
        out_shape=[
            jax.ShapeDtypeStruct((NP, DD), jnp.float32),
            jax.ShapeDtypeStruct((NP, 1), jnp.float32),
            jax.ShapeDtypeStruct((NP, 1), jnp.float32),
        ],
    )(x_pad, W, a_src[None, :], a_dst[None, :])


def _edge_logits_body(src_h, dst_h, as_h, ad_h, e_h, mx_h,
                      src_v, dst_v, as_v, ad_v, e_v, m_v):
    cid = lax.axis_index("c")
    sid = lax.axis_index("s")
    wid = sid * 2 + cid
    base = wid * CPT
    pltpu.sync_copy(src_h.at[pl.ds(base, CPT)], src_v)
    pltpu.sync_copy(dst_h.at[pl.ds(base, CPT)], dst_v)
    pltpu.sync_copy(as_h, as_v)
    pltpu.sync_copy(ad_h, ad_v)

    def initb(i, carry):
        m_v[pl.ds(i * 16, 16)] = jnp.full((16,), -jnp.inf, jnp.float32)
        return carry
    lax.fori_loop(0, NP // 16, initb, 0)

    def step(i, carry):
        off = i * 16
        s16 = src_v[pl.ds(off, 16)]
        d16 = dst_v[pl.ds(off, 16)]
        e16 = plsc.load_gather(as_v, [s16]) + plsc.load_gather(ad_v, [d16])
        e16 = jnp.where(e16 >= 0.0, e16, e16 * jnp.float32(0.2))
        e_v[pl.ds(off, 16)] = e16

        def wcond(need):
            return jnp.any(need)

        def wbody(need):
            plsc.store_scatter(m_v, [d16], e16, mask=need)
            cur = plsc.load_gather(m_v, [d16])
            return jnp.logical_and(need, e16 > cur)

        cur0 = plsc.load_gather(m_v, [d16])
        lax.while_loop(wcond, wbody, e16 > cur0)
        return carry
    lax.fori_loop(0, VPT, step, 0)

    pltpu.sync_copy(e_v, e_h.at[pl.ds(base, CPT)])
    pltpu.sync_copy(m_v, mx_h.at[wid])


_edge_logits = pl.kernel(
    _edge_logits_body,
    out_type=(
        jax.ShapeDtypeStruct((ETP,), jnp.float32),
        jax.ShapeDtypeStruct((NW, NP), jnp.float32),
    ),
    mesh=_mesh,
    scratch_types=[
        pltpu.VMEM((CPT,), jnp.int32),
        pltpu.VMEM((CPT,), jnp.int32),
        pltpu.VMEM((NP,), jnp.float32),
        pltpu.VMEM((NP,), jnp.float32),
        pltpu.VMEM((CPT,), jnp.float32),
        pltpu.VMEM((NP,), jnp.float32),
    ],
    compiler_params=pltpu.CompilerParams(needs_layout_passes=False),
)


def _combine_body(mx_ref, m_ref):
    m = jnp.max(mx_ref[...], axis=0, keepdims=True)
    m_ref[...] = jnp.where(jnp.isfinite(m), m, 0.0)


def _combine(maxes):
    return pl.pallas_call(
        _combine_body,
        out_shape=jax.ShapeDtypeStruct((1, NP), jnp.float32),
    )(maxes)


def _aggregate_body(src_h, dst_h, e_h, m_h, h_h, den_h, outp_h,
                    src1_v, dst1_v, e1_v, m_v, den_v, ex_v,
                    rows_v, zb_v, acc_sp, gsem):
    cid = lax.axis_index("c")
    sid = lax.axis_index("s")
    wid = sid * 2 + cid
    base = wid * CPT
    pltpu.sync_copy(m_h, m_v)

    zero16 = jnp.zeros((16,), jnp.float32)

    def zeroden(i, carry):
        den_v[pl.ds(i * 16, 16)] = zero16
        return carry
    lax.fori_loop(0, NP // 16, zeroden, 0)

    def zerozb(i, carry):
        r = i // 8
        c = i % 8
        zb_v[r, pl.ds(c * 16, 16)] = zero16
        return carry
    lax.fori_loop(0, 8 * 8, zerozb, 0)

    def zeroacc(k, carry):
        pltpu.sync_copy(zb_v, acc_sp.at[pl.ds(sid * RPT + k * 8, 8)])
        return carry
    lax.fori_loop(0, RPT // 8, zeroacc, 0)

    plsc.subcore_barrier()

    def chunk(g, carry):
        cbase = base + g * CK
        pltpu.sync_copy(src_h.at[pl.ds(cbase, CK)], src1_v)
        pltpu.sync_copy(dst_h.at[pl.ds(cbase, CK)], dst1_v)
        pltpu.sync_copy(e_h.at[pl.ds(cbase, CK)], e1_v)
        pltpu.async_copy(h_h.at[src1_v], rows_v, gsem).wait()

        def vec(v, c2):
            d16 = dst1_v[pl.ds(v * 16, 16)]
            e16 = e1_v[pl.ds(v * 16, 16)]
            mm = plsc.load_gather(m_v, [d16])
            ex = jnp.exp(e16 - mm)
            plsc.addupdate_scatter(den_v, [d16], ex)
            ex_v[...] = ex

            def edge(r, c3):
                lane = plsc.load_gather(ex_v, [jnp.full((16,), r, jnp.int32)])
                row = v * 16 + r
                for c8 in range(8):
                    sl = pl.ds(c8 * 16, 16)
                    rows_v[row, sl] = rows_v[row, sl] * lane
                return c3
            lax.fori_loop(0, 16, edge, c2)
            return c2
        lax.fori_loop(0, 8, vec, 0)

        pltpu.sync_copy(rows_v, acc_sp.at[dst1_v], add=True)
        return carry

    lax.fori_loop(0, NG, chunk, 0)

    plsc.subcore_barrier()
    pltpu.sync_copy(acc_sp.at[pl.ds(sid * RPT, RPT)],
                    outp_h.at[cid, pl.ds(sid * RPT, RPT)])
    pltpu.sync_copy(den_v, den_h.at[wid])


_aggregate = pl.kernel(
    _aggregate_body,
    out_type=(
        jax.ShapeDtypeStruct((NW, NP), jnp.float32),
        jax.ShapeDtypeStruct((2, NP, DD), jnp.float32),
    ),
    mesh=_mesh,
    scratch_types=[
        pltpu.VMEM((CK,), jnp.int32),
        pltpu.VMEM((CK,), jnp.int32),
        pltpu.VMEM((CK,), jnp.float32),
        pltpu.VMEM((NP,), jnp.float32),
        pltpu.VMEM((NP,), jnp.float32),
        pltpu.VMEM((16,), jnp.float32),
        pltpu.VMEM((CK, DD), jnp.float32),
        pltpu.VMEM((8, DD), jnp.float32),
        pltpu.VMEM_SHARED((NP, DD), jnp.float32),
        pltpu.SemaphoreType.DMA,
    ],
    compiler_params=pltpu.CompilerParams(needs_layout_passes=False),
)


def _final_body(outp_ref, den_ref, batch_ref, bias_ref, wl_ref, bl_ref,
                y_ref, pool, cnt):
    i = pl.program_id(0)

    @pl.when(i == 0)
    def _init():
        pool[...] = jnp.zeros((GG, DD), jnp.float32)
        cnt[...] = jnp.zeros((GG, DD), jnp.float32)

    ds = jnp.sum(den_ref[...], axis=0)
    acc = outp_ref[0] + outp_ref[1]
    rows = acc / (ds + 1e-16)[:, None] + bias_ref[...]
    b = batch_ref[0, 0, :]
    M = (lax.broadcasted_iota(jnp.int32, (GG, 1280), 0) == b[None, :]
         ).astype(jnp.float32)
    pool[...] += jnp.dot(M, rows, preferred_element_type=jnp.float32)
    cnt[...] += jnp.dot(M, jnp.ones((1280, DD), jnp.float32),
                        preferred_element_type=jnp.float32)

    @pl.when(i == NP // 1280 - 1)
    def _fin():
        pooled = pool[...] / jnp.maximum(cnt[...], 1.0)
        y = jnp.dot(pooled, wl_ref[...], preferred_element_type=jnp.float32)
        y_ref[...] = jnp.maximum(y + bl_ref[...], 0.0)


def _final(outp, denoms, batch3, bias, W_lin, b_lin):
    nb = NP // 1280
    return pl.pallas_call(
        _final_body,
        grid=(nb,),
        in_specs=[
            pl.BlockSpec((2, 1280, DD), lambda i: (0, i, 0)),
            pl.BlockSpec((NW, 1280), lambda i: (0, i)),
            pl.BlockSpec((1, 1, 1280), lambda i: (i, 0, 0)),
            pl.BlockSpec((1, DD), lambda i: (0, 0)),
            pl.BlockSpec((DD, NOUT), lambda i: (0, 0)),
            pl.BlockSpec((1, NOUT), lambda i: (0, 0)),
        ],
        out_specs=pl.BlockSpec((GG, NOUT), lambda i: (0, 0)),
        out_shape=jax.ShapeDtypeStruct((GG, NOUT), jnp.float32),
        scratch_shapes=[
            pltpu.VMEM((GG, DD), jnp.float32),
            pltpu.VMEM((GG, DD), jnp.float32),
        ],
    )(outp, denoms, batch3, bias[None, :], W_lin, b_lin[None, :])


def kernel(x, edge_index, batch, W, a_src, a_dst, bias, W_lin, b_lin):
    loop = jnp.arange(NN, dtype=jnp.int32)
    pad = jnp.full((ETP - ET,), NN, jnp.int32)
    src = jnp.concatenate([edge_index[0], loop, pad])
    dst = jnp.concatenate([edge_index[1], loop, pad])
    x_pad = jnp.pad(x, ((0, NP - NN), (0, 0)))
    batch_pad = jnp.pad(batch, (0, NP - NN), constant_values=GG)
    batch3 = batch_pad.reshape(NP // 1280, 1, 1280)

    h, s2, d2 = _prep(x_pad, W, a_src, a_dst)
    as_pad = s2[:, 0]
    ad_pad = d2[:, 0]

    e, maxes = _edge_logits(src, dst, as_pad, ad_pad)
    m = _combine(maxes)[0]
    denoms, outp = _aggregate(src, dst, e, m, h)
    return _final(outp, denoms, batch3, bias, W_lin, b_lin)


# speedup vs baseline: 26.0938x; 1.2583x over previous
import functools

import jax
import jax.numpy as jnp
from jax import lax
from jax.experimental import pallas as pl
from jax.experimental.pallas import tpu as pltpu
from jax.experimental.pallas import tpu_sc as plsc

NN = 10000
DD = 128
GG = 64
NOUT = 10
NP = 10240
EE = 320000
ET = EE + NN
NW = 32
CK = 96
NG = 108
CPT = NG * CK
ETP = NW * CPT
VPT = CPT // 16
RPT = NP // 16

_mesh = plsc.VectorSubcoreMesh(core_axis_name="c", subcore_axis_name="s")


def _prep_body(x_ref, w_ref, as_ref, ad_ref, h_ref, s_ref, d_ref):
    h = jnp.dot(x_ref[...], w_ref[...], preferred_element_type=jnp.float32)
    h_ref[...] = h
    s_ref[...] = jnp.sum(h * as_ref[...], axis=-1, keepdims=True)
    d_ref[...] = jnp.sum(h * ad_ref[...], axis=-1, keepdims=True)


def _prep(x_pad, W, a_src, a_dst):
    nb = NP // 1280
    return pl.pallas_call(
        _prep_body,
        grid=(nb,),
        in_specs=[
            pl.BlockSpec((1280, DD), lambda i: (i, 0)),
            pl.BlockSpec((DD, DD), lambda i: (0, 0)),
            pl.BlockSpec((1, DD), lambda i: (0, 0)),
            pl.BlockSpec((1, DD), lambda i: (0, 0)),
        ],
        out_specs=[
            pl.BlockSpec((1280, DD), lambda i: (i, 0)),
            pl.BlockSpec((1280, 1), lambda i: (i, 0)),
            pl.BlockSpec((1280, 1), lambda i: (i, 0)),
        ],
        out_shape=[
            jax.ShapeDtypeStruct((NP, DD), jnp.float32),
            jax.ShapeDtypeStruct((NP, 1), jnp.float32),
            jax.ShapeDtypeStruct((NP, 1), jnp.float32),
        ],
    )(x_pad, W, a_src[None, :], a_dst[None, :])


def _edge_logits_body(src_h, dst_h, as_h, ad_h, e_h, mx_h,
                      src_v, dst_v, as_v, ad_v, e_v, m_v):
    cid = lax.axis_index("c")
    sid = lax.axis_index("s")
    wid = sid * 2 + cid
    base = wid * CPT
    pltpu.sync_copy(src_h.at[pl.ds(base, CPT)], src_v)
    pltpu.sync_copy(dst_h.at[pl.ds(base, CPT)], dst_v)
    pltpu.sync_copy(as_h, as_v)
    pltpu.sync_copy(ad_h, ad_v)

    def initb(i, carry):
        m_v[pl.ds(i * 16, 16)] = jnp.full((16,), -jnp.inf, jnp.float32)
        return carry
    lax.fori_loop(0, NP // 16, initb, 0)

    def step(i, carry):
        off = i * 16
        s16 = src_v[pl.ds(off, 16)]
        d16 = dst_v[pl.ds(off, 16)]
        e16 = plsc.load_gather(as_v, [s16]) + plsc.load_gather(ad_v, [d16])
        e16 = jnp.where(e16 >= 0.0, e16, e16 * jnp.float32(0.2))
        e_v[pl.ds(off, 16)] = e16

        def wcond(need):
            return jnp.any(need)

        def wbody(need):
            plsc.store_scatter(m_v, [d16], e16, mask=need)
            cur = plsc.load_gather(m_v, [d16])
            return jnp.logical_and(need, e16 > cur)

        cur0 = plsc.load_gather(m_v, [d16])
        lax.while_loop(wcond, wbody, e16 > cur0)
        return carry
    lax.fori_loop(0, VPT, step, 0)

    pltpu.sync_copy(e_v, e_h.at[pl.ds(base, CPT)])
    pltpu.sync_copy(m_v, mx_h.at[wid])


_edge_logits = pl.kernel(
    _edge_logits_body,
    out_type=(
        jax.ShapeDtypeStruct((ETP,), jnp.float32),
        jax.ShapeDtypeStruct((NW, NP), jnp.float32),
    ),
    mesh=_mesh,
    scratch_types=[
        pltpu.VMEM((CPT,), jnp.int32),
        pltpu.VMEM((CPT,), jnp.int32),
        pltpu.VMEM((NP,), jnp.float32),
        pltpu.VMEM((NP,), jnp.float32),
        pltpu.VMEM((CPT,), jnp.float32),
        pltpu.VMEM((NP,), jnp.float32),
    ],
    compiler_params=pltpu.CompilerParams(needs_layout_passes=False),
)


def _combine_body(mx_ref, m_ref):
    m = jnp.max(mx_ref[...], axis=0, keepdims=True)
    m_ref[...] = jnp.where(jnp.isfinite(m), m, 0.0)


def _combine(maxes):
    return pl.pallas_call(
        _combine_body,
        out_shape=jax.ShapeDtypeStruct((1, NP), jnp.float32),
    )(maxes)


def _aggregate_body_real(p_h, m_h, h_h, den_h, outp_h,
                         pA, pB, m_v, den_v, ex_v, rowsA, rowsB, acc_sp,
                         gsA, gsB, ssA, ssB):
    cid = lax.axis_index("c")
    sid = lax.axis_index("s")
    wid = sid * 2 + cid
    pltpu.sync_copy(m_h, m_v)

    zero16 = jnp.zeros((16,), jnp.float32)

    def zeroden(i, carry):
        den_v[pl.ds(i * 16, 16)] = zero16
        return carry
    lax.fori_loop(0, NP // 16, zeroden, 0)

    def zerorows(i, carry):
        rowsA[i // 8, pl.ds((i % 8) * 16, 16)] = zero16
        return carry
    lax.fori_loop(0, CK * 8, zerorows, 0)

    def zeroacc(k, carry):
        pltpu.sync_copy(rowsA, acc_sp.at[pl.ds(sid * RPT + k * CK, CK)])
        return carry
    lax.fori_loop(0, RPT // CK, zeroacc, 0)
    pltpu.sync_copy(rowsA.at[pl.ds(0, RPT % CK)],
                    acc_sp.at[pl.ds(sid * RPT + (RPT // CK) * CK, RPT % CK)])

    plsc.subcore_barrier()

    def gather_cp(pX, rowsX, gsX):
        return pltpu.make_async_copy(h_h.at[pX.at[0]], rowsX, gsX)

    def scatter_cp(pX, rowsX, ssX):
        return pltpu.async_copy(rowsX, acc_sp.at[pX.at[1]], ssX, add=True)

    def scatter_wait(pX, rowsX, ssX):
        d = pltpu.make_async_copy(rowsX, acc_sp.at[pX.at[1]], ssX)
        d.wait()

    def compute(pX, rowsX):
        def vec(v, carry):
            d16 = pX[1, pl.ds(v * 16, 16)]
            e16 = plsc.bitcast(pX[2, pl.ds(v * 16, 16)], jnp.float32)
            mm = plsc.load_gather(m_v, [d16])
            ex = jnp.exp(e16 - mm)
            plsc.addupdate_scatter(den_v, [d16], ex)
            ex_v[...] = ex
            for r in range(16):
                lane = plsc.load_gather(ex_v, [jnp.full((16,), r, jnp.int32)])
                row = v * 16 + r
                for c8 in range(8):
                    sl = pl.ds(c8 * 16, 16)
                    rowsX[row, sl] = rowsX[row, sl] * lane
            return carry
        lax.fori_loop(0, CK // 16, vec, 0)

    pltpu.sync_copy(p_h.at[wid, 0], pA)
    gather_cp(pA, rowsA, gsA).start()

    def half(p, carry):
        gather_cp(pA, rowsA, gsA).wait()

        @pl.when(p > 0)
        def _wB():
            scatter_wait(pB, rowsB, ssB)
        pltpu.sync_copy(p_h.at[wid, 2 * p + 1], pB)
        gather_cp(pB, rowsB, gsB).start()
        compute(pA, rowsA)
        scatter_cp(pA, rowsA, ssA)

        gather_cp(pB, rowsB, gsB).wait()

        @pl.when(p < NG // 2 - 1)
        def _nextA():
            scatter_wait(pA, rowsA, ssA)
            pltpu.sync_copy(p_h.at[wid, 2 * p + 2], pA)
            gather_cp(pA, rowsA, gsA).start()
        compute(pB, rowsB)
        scatter_cp(pB, rowsB, ssB)
        return carry

    lax.fori_loop(0, NG // 2, half, 0)
    scatter_wait(pA, rowsA, ssA)
    scatter_wait(pB, rowsB, ssB)

    plsc.subcore_barrier()
    pltpu.sync_copy(acc_sp.at[pl.ds(sid * RPT, RPT)],
                    outp_h.at[cid, pl.ds(sid * RPT, RPT)])
    pltpu.sync_copy(den_v, den_h.at[wid])


_aggregate = pl.kernel(
    _aggregate_body_real,
    out_type=(
        jax.ShapeDtypeStruct((NW, NP), jnp.float32),
        jax.ShapeDtypeStruct((2, NP, DD), jnp.float32),
    ),
    mesh=_mesh,
    scratch_types=[
        pltpu.VMEM((3, CK), jnp.int32),
        pltpu.VMEM((3, CK), jnp.int32),
        pltpu.VMEM((NP,), jnp.float32),
        pltpu.VMEM((NP,), jnp.float32),
        pltpu.VMEM((16,), jnp.float32),
        pltpu.VMEM((CK, DD), jnp.float32),
        pltpu.VMEM((CK, DD), jnp.float32),
        pltpu.VMEM_SHARED((NP, DD), jnp.float32),
        pltpu.SemaphoreType.DMA,
        pltpu.SemaphoreType.DMA,
        pltpu.SemaphoreType.DMA,
        pltpu.SemaphoreType.DMA,
    ],
    compiler_params=pltpu.CompilerParams(needs_layout_passes=False),
)


def _final_body(outp_ref, den_ref, batch_ref, bias_ref, wl_ref, bl_ref,
                y_ref, pool, cnt):
    i = pl.program_id(0)

    @pl.when(i == 0)
    def _init():
        pool[...] = jnp.zeros((GG, DD), jnp.float32)
        cnt[...] = jnp.zeros((GG, DD), jnp.float32)

    ds = jnp.sum(den_ref[...], axis=0)
    acc = outp_ref[0] + outp_ref[1]
    rows = acc / (ds + 1e-16)[:, None] + bias_ref[...]
    b = batch_ref[0, 0, :]
    M = (lax.broadcasted_iota(jnp.int32, (GG, 1280), 0) == b[None, :]
         ).astype(jnp.float32)
    pool[...] += jnp.dot(M, rows, preferred_element_type=jnp.float32)
    cnt[...] += jnp.dot(M, jnp.ones((1280, DD), jnp.float32),
                        preferred_element_type=jnp.float32)

    @pl.when(i == NP // 1280 - 1)
    def _fin():
        pooled = pool[...] / jnp.maximum(cnt[...], 1.0)
        y = jnp.dot(pooled, wl_ref[...], preferred_element_type=jnp.float32)
        y_ref[...] = jnp.maximum(y + bl_ref[...], 0.0)


def _final(outp, denoms, batch3, bias, W_lin, b_lin):
    nb = NP // 1280
    return pl.pallas_call(
        _final_body,
        grid=(nb,),
        in_specs=[
            pl.BlockSpec((2, 1280, DD), lambda i: (0, i, 0)),
            pl.BlockSpec((NW, 1280), lambda i: (0, i)),
            pl.BlockSpec((1, 1, 1280), lambda i: (i, 0, 0)),
            pl.BlockSpec((1, DD), lambda i: (0, 0)),
            pl.BlockSpec((DD, NOUT), lambda i: (0, 0)),
            pl.BlockSpec((1, NOUT), lambda i: (0, 0)),
        ],
        out_specs=pl.BlockSpec((GG, NOUT), lambda i: (0, 0)),
        out_shape=jax.ShapeDtypeStruct((GG, NOUT), jnp.float32),
        scratch_shapes=[
            pltpu.VMEM((GG, DD), jnp.float32),
            pltpu.VMEM((GG, DD), jnp.float32),
        ],
    )(outp, denoms, batch3, bias[None, :], W_lin, b_lin[None, :])


def kernel(x, edge_index, batch, W, a_src, a_dst, bias, W_lin, b_lin):
    loop = jnp.arange(NN, dtype=jnp.int32)
    pad = jnp.full((ETP - ET,), NN, jnp.int32)
    src = jnp.concatenate([edge_index[0], loop, pad])
    dst = jnp.concatenate([edge_index[1], loop, pad])
    x_pad = jnp.pad(x, ((0, NP - NN), (0, 0)))
    batch_pad = jnp.pad(batch, (0, NP - NN), constant_values=GG)
    batch3 = batch_pad.reshape(NP // 1280, 1, 1280)

    h, s2, d2 = _prep(x_pad, W, a_src, a_dst)
    as_pad = s2[:, 0]
    ad_pad = d2[:, 0]

    e, maxes = _edge_logits(src, dst, as_pad, ad_pad)
    m = _combine(maxes)[0]
    ebits = lax.bitcast_convert_type(e, jnp.int32)
    P = jnp.stack([src.reshape(NW, NG, CK), dst.reshape(NW, NG, CK),
                   ebits.reshape(NW, NG, CK)], axis=2)
    denoms, outp = _aggregate(P, m, h)
    return _final(outp, denoms, batch3, bias, W_lin, b_lin)


# speedup vs baseline: 31.0192x; 1.1888x over previous
import functools

import jax
import jax.numpy as jnp
from jax import lax
from jax.experimental import pallas as pl
from jax.experimental.pallas import tpu as pltpu
from jax.experimental.pallas import tpu_sc as plsc

NN = 10000
DD = 128
GG = 64
NOUT = 10
NP = 10240
EE = 320000
ET = EE + NN
NW = 32
CK = 128
NG = 82
CPT = NG * CK
ETP = NW * CPT
VPT = CPT // 16
RPT = NP // 16

_mesh = plsc.VectorSubcoreMesh(core_axis_name="c", subcore_axis_name="s")


def _prep_body(x_ref, w_ref, as_ref, ad_ref, h_ref, s_ref, d_ref):
    h = jnp.dot(x_ref[...], w_ref[...], preferred_element_type=jnp.float32)
    h_ref[...] = h.astype(jnp.bfloat16)
    s_ref[...] = jnp.sum(h * as_ref[...], axis=-1, keepdims=True)
    d_ref[...] = jnp.sum(h * ad_ref[...], axis=-1, keepdims=True)


def _prep(x_pad, W, a_src, a_dst):
    nb = NP // 1280
    return pl.pallas_call(
        _prep_body,
        grid=(nb,),
        in_specs=[
            pl.BlockSpec((1280, DD), lambda i: (i, 0)),
            pl.BlockSpec((DD, DD), lambda i: (0, 0)),
            pl.BlockSpec((1, DD), lambda i: (0, 0)),
            pl.BlockSpec((1, DD), lambda i: (0, 0)),
        ],
        out_specs=[
            pl.BlockSpec((1280, DD), lambda i: (i, 0)),
            pl.BlockSpec((1280, 1), lambda i: (i, 0)),
            pl.BlockSpec((1280, 1), lambda i: (i, 0)),
        ],
        out_shape=[
            jax.ShapeDtypeStruct((NP, DD), jnp.bfloat16),
            jax.ShapeDtypeStruct((NP, 1), jnp.float32),
            jax.ShapeDtypeStruct((NP, 1), jnp.float32),
        ],
    )(x_pad, W, a_src[None, :], a_dst[None, :])


def _edge_logits_body(src_h, dst_h, as_h, ad_h, e_h, mx_h,
                      src_v, dst_v, as_v, ad_v, e_v, m_v):
    cid = lax.axis_index("c")
    sid = lax.axis_index("s")
    wid = sid * 2 + cid
    base = wid * CPT
    pltpu.sync_copy(src_h.at[pl.ds(base, CPT)], src_v)
    pltpu.sync_copy(dst_h.at[pl.ds(base, CPT)], dst_v)
    pltpu.sync_copy(as_h, as_v)
    pltpu.sync_copy(ad_h, ad_v)

    def initb(i, carry):
        m_v[pl.ds(i * 16, 16)] = jnp.full((16,), -jnp.inf, jnp.float32)
        return carry
    lax.fori_loop(0, NP // 16, initb, 0)

    def step(i, carry):
        off = i * 16
        s16 = src_v[pl.ds(off, 16)]
        d16 = dst_v[pl.ds(off, 16)]
        e16 = plsc.load_gather(as_v, [s16]) + plsc.load_gather(ad_v, [d16])
        e16 = jnp.where(e16 >= 0.0, e16, e16 * jnp.float32(0.2))
        e_v[pl.ds(off, 16)] = e16

        def wcond(need):
            return jnp.any(need)

        def wbody(need):
            plsc.store_scatter(m_v, [d16], e16, mask=need)
            cur = plsc.load_gather(m_v, [d16])
            return jnp.logical_and(need, e16 > cur)

        cur0 = plsc.load_gather(m_v, [d16])
        lax.while_loop(wcond, wbody, e16 > cur0)
        return carry
    lax.fori_loop(0, VPT, step, 0)

    pltpu.sync_copy(e_v, e_h.at[pl.ds(base, CPT)])
    pltpu.sync_copy(m_v, mx_h.at[wid])


_edge_logits = pl.kernel(
    _edge_logits_body,
    out_type=(
        jax.ShapeDtypeStruct((ETP,), jnp.float32),
        jax.ShapeDtypeStruct((NW, NP), jnp.float32),
    ),
    mesh=_mesh,
    scratch_types=[
        pltpu.VMEM((CPT,), jnp.int32),
        pltpu.VMEM((CPT,), jnp.int32),
        pltpu.VMEM((NP,), jnp.float32),
        pltpu.VMEM((NP,), jnp.float32),
        pltpu.VMEM((CPT,), jnp.float32),
        pltpu.VMEM((NP,), jnp.float32),
    ],
    compiler_params=pltpu.CompilerParams(needs_layout_passes=False),
)


def _combine_body(mx_ref, m_ref):
    m = jnp.max(mx_ref[...], axis=0, keepdims=True)
    m_ref[...] = jnp.where(jnp.isfinite(m), m, 0.0)


def _combine(maxes):
    return pl.pallas_call(
        _combine_body,
        out_shape=jax.ShapeDtypeStruct((1, NP), jnp.float32),
    )(maxes)


def _aggregate_body_real(p_h, dst_h, m_h, h_h, den_h, outp_h,
                         pA, pB, dA, dB, m_v, den_v, rowsA, rowsB, acc_sp,
                         gsA, gsB, ssA, ssB, stA, stB):
    cid = lax.axis_index("c")
    sid = lax.axis_index("s")
    wid = sid * 2 + cid
    pltpu.sync_copy(m_h, m_v)

    zero16 = jnp.zeros((16,), jnp.float32)

    def zeroden(i, carry):
        den_v[pl.ds(i * 16, 16)] = zero16
        return carry
    lax.fori_loop(0, NP // 16, zeroden, 0)

    zero32 = jnp.zeros((32,), jnp.bfloat16)

    def zerorows(i, carry):
        rowsA[i // 4, pl.ds((i % 4) * 32, 32)] = zero32
        return carry
    lax.fori_loop(0, CK * 4, zerorows, 0)

    def zeroacc(k, carry):
        pltpu.sync_copy(rowsA, acc_sp.at[pl.ds(sid * RPT + k * CK, CK)])
        return carry
    lax.fori_loop(0, RPT // CK, zeroacc, 0)
    if RPT % CK:
        pltpu.sync_copy(rowsA.at[pl.ds(0, RPT % CK)],
                        acc_sp.at[pl.ds(sid * RPT + (RPT // CK) * CK, RPT % CK)])

    plsc.subcore_barrier()

    HK = CK // 2

    def gather_cp(pX, rowsX, gsX):
        lo = pltpu.make_async_copy(
            h_h.at[pX.at[0, pl.ds(0, HK)]], rowsX.at[pl.ds(0, HK)], gsX)
        hi = pltpu.make_async_copy(
            h_h.at[pX.at[0, pl.ds(HK, HK)]], rowsX.at[pl.ds(HK, HK)], gsX)
        return lo, hi


    def stage_cp(g, pX, stX):
        return pltpu.make_async_copy(p_h.at[wid, g], pX, stX)

    def scatter_cp(dX, rowsX, ssX):
        return pltpu.async_copy(rowsX, acc_sp.at[dX], ssX, add=True)

    def scatter_wait(dX, rowsX, ssX):
        pltpu.make_async_copy(rowsX, acc_sp.at[dX], ssX).wait()

    def dcopy(pX, dX):
        for i in range(CK // 16):
            dX[pl.ds(i * 16, 16)] = pX[1, pl.ds(i * 16, 16)]

    def compute(pX, rowsX):
        def vec(v, carry):
            d16 = pX[1, pl.ds(v * 16, 16)]
            e16 = plsc.bitcast(pX[2, pl.ds(v * 16, 16)], jnp.float32)
            mm = plsc.load_gather(m_v, [d16])
            ex = jnp.exp(e16 - mm)
            plsc.addupdate_scatter(den_v, [d16], ex)
            gdn = lax.GatherDimensionNumbers(
                offset_dims=(), collapsed_slice_dims=(0,),
                start_index_map=(0,))
            for r in range(16):
                lane = lax.gather(
                    ex, jnp.full((16, 1), r, jnp.int32), gdn, (1,),
                    mode=lax.GatherScatterMode.PROMISE_IN_BOUNDS)
                row = v * 16 + r
                for c8 in range(8):
                    sl = pl.ds(c8 * 16, 16)
                    rowsX[row, sl] = rowsX[row, sl] * lane
            return carry
        lax.fori_loop(0, CK // 16, vec, 0)

    NH = NG // 2
    pltpu.sync_copy(p_h.at[wid, 0], pA)
    stage_cp(1, pB, stB).start()
    for _c in gather_cp(pA, rowsA, gsA):
        _c.start()

    def half(p, carry):
        for _c in gather_cp(pA, rowsA, gsA):
            _c.wait()

        @pl.when(p > 0)
        def _freeB():
            scatter_wait(dB, rowsB, ssB)
        stage_cp(2 * p + 1, pB, stB).wait()
        for _c in gather_cp(pB, rowsB, gsB):
            _c.start()
        compute(pA, rowsA)
        dcopy(pA, dA)
        scatter_cp(dA, rowsA, ssA)

        @pl.when(p < NH - 1)
        def _stA():
            stage_cp(2 * p + 2, pA, stA).start()

        for _c in gather_cp(pB, rowsB, gsB):
            _c.wait()
        scatter_wait(dA, rowsA, ssA)

        @pl.when(p < NH - 1)
        def _gA():
            stage_cp(2 * p + 2, pA, stA).wait()
            for _c in gather_cp(pA, rowsA, gsA):
                _c.start()
        compute(pB, rowsB)
        dcopy(pB, dB)
        scatter_cp(dB, rowsB, ssB)

        @pl.when(p < NH - 1)
        def _stB():
            stage_cp(2 * p + 3, pB, stB).start()
        return carry

    lax.fori_loop(0, NH, half, 0)
    scatter_wait(dB, rowsB, ssB)

    plsc.subcore_barrier()
    pltpu.sync_copy(acc_sp.at[pl.ds(sid * RPT, RPT)],
                    outp_h.at[cid, pl.ds(sid * RPT, RPT)])
    pltpu.sync_copy(den_v, den_h.at[wid])


_aggregate = pl.kernel(
    _aggregate_body_real,
    out_type=(
        jax.ShapeDtypeStruct((NW, NP), jnp.float32),
        jax.ShapeDtypeStruct((2, NP, DD), jnp.bfloat16),
    ),
    mesh=_mesh,
    scratch_types=[
        pltpu.VMEM((3, CK), jnp.int32),
        pltpu.VMEM((3, CK), jnp.int32),
        pltpu.VMEM((CK,), jnp.int32),
        pltpu.VMEM((CK,), jnp.int32),
        pltpu.VMEM((NP,), jnp.float32),
        pltpu.VMEM((NP,), jnp.float32),
        pltpu.VMEM((CK, DD), jnp.bfloat16),
        pltpu.VMEM((CK, DD), jnp.bfloat16),
        pltpu.VMEM_SHARED((NP, DD), jnp.bfloat16),
        pltpu.SemaphoreType.DMA,
        pltpu.SemaphoreType.DMA,
        pltpu.SemaphoreType.DMA,
        pltpu.SemaphoreType.DMA,
        pltpu.SemaphoreType.DMA,
        pltpu.SemaphoreType.DMA,
    ],
    compiler_params=pltpu.CompilerParams(needs_layout_passes=False),
)


def _final_body(outp_ref, den_ref, batch_ref, bias_ref, wl_ref, bl_ref,
                y_ref, pool, cnt):
    i = pl.program_id(0)

    @pl.when(i == 0)
    def _init():
        pool[...] = jnp.zeros((GG, DD), jnp.float32)
        cnt[...] = jnp.zeros((GG, DD), jnp.float32)

    ds = jnp.sum(den_ref[...], axis=0)
    acc = (outp_ref[0].astype(jnp.float32)
           + outp_ref[1].astype(jnp.float32))
    rows = acc / (ds + 1e-16)[:, None] + bias_ref[...]
    b = batch_ref[0, 0, :]
    M = (lax.broadcasted_iota(jnp.int32, (GG, 1280), 0) == b[None, :]
         ).astype(jnp.float32)
    pool[...] += jnp.dot(M, rows, preferred_element_type=jnp.float32)
    cnt[...] += jnp.dot(M, jnp.ones((1280, DD), jnp.float32),
                        preferred_element_type=jnp.float32)

    @pl.when(i == NP // 1280 - 1)
    def _fin():
        pooled = pool[...] / jnp.maximum(cnt[...], 1.0)
        y = jnp.dot(pooled, wl_ref[...], preferred_element_type=jnp.float32)
        y_ref[...] = jnp.maximum(y + bl_ref[...], 0.0)


def _final(outp, denoms, batch3, bias, W_lin, b_lin):
    nb = NP // 1280
    return pl.pallas_call(
        _final_body,
        grid=(nb,),
        in_specs=[
            pl.BlockSpec((2, 1280, DD), lambda i: (0, i, 0)),
            pl.BlockSpec((NW, 1280), lambda i: (0, i)),
            pl.BlockSpec((1, 1, 1280), lambda i: (i, 0, 0)),
            pl.BlockSpec((1, DD), lambda i: (0, 0)),
            pl.BlockSpec((DD, NOUT), lambda i: (0, 0)),
            pl.BlockSpec((1, NOUT), lambda i: (0, 0)),
        ],
        out_specs=pl.BlockSpec((GG, NOUT), lambda i: (0, 0)),
        out_shape=jax.ShapeDtypeStruct((GG, NOUT), jnp.float32),
        scratch_shapes=[
            pltpu.VMEM((GG, DD), jnp.float32),
            pltpu.VMEM((GG, DD), jnp.float32),
        ],
    )(outp, denoms, batch3, bias[None, :], W_lin, b_lin[None, :])


def kernel(x, edge_index, batch, W, a_src, a_dst, bias, W_lin, b_lin):
    loop = jnp.arange(NN, dtype=jnp.int32)
    pad = jnp.full((ETP - ET,), NN, jnp.int32)
    src = jnp.concatenate([edge_index[0], loop, pad])
    dst = jnp.concatenate([edge_index[1], loop, pad])
    x_pad = jnp.pad(x, ((0, NP - NN), (0, 0)))
    batch_pad = jnp.pad(batch, (0, NP - NN), constant_values=GG)
    batch3 = batch_pad.reshape(NP // 1280, 1, 1280)

    h, s2, d2 = _prep(x_pad, W, a_src, a_dst)
    as_pad = s2[:, 0]
    ad_pad = d2[:, 0]

    e, maxes = _edge_logits(src, dst, as_pad, ad_pad)
    m = _combine(maxes)[0]
    ebits = lax.bitcast_convert_type(e, jnp.int32)
    P = jnp.stack([src.reshape(NW, NG, CK), dst.reshape(NW, NG, CK),
                   ebits.reshape(NW, NG, CK)], axis=2)
    denoms, outp = _aggregate(P, dst, m, h)
    return _final(outp, denoms, batch3, bias, W_lin, b_lin)


# speedup vs baseline: 31.4604x; 1.0142x over previous
import jax
import jax.numpy as jnp
from jax import lax
from jax.experimental import pallas as pl
from jax.experimental.pallas import tpu as pltpu
from jax.experimental.pallas import tpu_sc as plsc

NN = 10000
DD = 128
GG = 64
NOUT = 10
NP = 10240
EE = 320000
ET = EE + NN
NW = 32
CK = 96
NG = 108
CPT = NG * CK
ETP = NW * CPT
VPT = CPT // 16
RPT = NP // 16

_mesh = plsc.VectorSubcoreMesh(core_axis_name="c", subcore_axis_name="s")


def _prep_body(x_ref, w_ref, as_ref, ad_ref, h_ref, s_ref, d_ref):
    h = jnp.dot(x_ref[...], w_ref[...], preferred_element_type=jnp.float32)
    h_ref[...] = h
    s_ref[...] = jnp.sum(h * as_ref[...], axis=-1, keepdims=True)
    d_ref[...] = jnp.sum(h * ad_ref[...], axis=-1, keepdims=True)


def _prep(x, W, a_src, a_dst):
    nb = NN // 1000
    return pl.pallas_call(
        _prep_body,
        grid=(nb,),
        in_specs=[
            pl.BlockSpec((1000, DD), lambda i: (i, 0)),
            pl.BlockSpec((DD, DD), lambda i: (0, 0)),
            pl.BlockSpec((1, DD), lambda i: (0, 0)),
            pl.BlockSpec((1, DD), lambda i: (0, 0)),
        ],
        out_specs=[
            pl.BlockSpec((1000, DD), lambda i: (i, 0)),
            pl.BlockSpec((1000, 1), lambda i: (i, 0)),
            pl.BlockSpec((1000, 1), lambda i: (i, 0)),
        ],
        out_shape=[
            jax.ShapeDtypeStruct((NN, DD), jnp.float32),
            jax.ShapeDtypeStruct((NN, 1), jnp.float32),
            jax.ShapeDtypeStruct((NN, 1), jnp.float32),
        ],
    )(x, W, a_src[None, :], a_dst[None, :])


def _edge_logits_body(src_h, dst_h, as_h, ad_h, e_h, mx_h,
                      src_v, dst_v, as_v, ad_v, e_v, m_v):
    cid = lax.axis_index("c")
    sid = lax.axis_index("s")
    wid = sid * 2 + cid
    base = wid * CPT
    pltpu.sync_copy(src_h.at[pl.ds(base, CPT)], src_v)
    pltpu.sync_copy(dst_h.at[pl.ds(base, CPT)], dst_v)
    pltpu.sync_copy(as_h, as_v)
    pltpu.sync_copy(ad_h, ad_v)

    def initb(i, carry):
        m_v[pl.ds(i * 16, 16)] = jnp.full((16,), -jnp.inf, jnp.float32)
        return carry
    lax.fori_loop(0, NP // 16, initb, 0)

    def step(i, carry):
        off = i * 16
        s16 = src_v[pl.ds(off, 16)]
        d16 = dst_v[pl.ds(off, 16)]
        e16 = plsc.load_gather(as_v, [s16]) + plsc.load_gather(ad_v, [d16])
        e16 = jnp.where(e16 >= 0.0, e16, e16 * jnp.float32(0.2))
        flat = base + off + lax.iota(jnp.int32, 16)
        e16 = jnp.where(flat < ET, e16, jnp.float32(-1e30))
        e_v[pl.ds(off, 16)] = e16

        def wcond(need):
            return jnp.any(need)

        def wbody(need):
            plsc.store_scatter(m_v, [d16], e16, mask=need)
            cur = plsc.load_gather(m_v, [d16])
            return jnp.logical_and(need, e16 > cur)

        cur0 = plsc.load_gather(m_v, [d16])
        lax.while_loop(wcond, wbody, e16 > cur0)
        return carry
    lax.fori_loop(0, VPT, step, 0)

    pltpu.sync_copy(e_v, e_h.at[pl.ds(base, CPT)])
    pltpu.sync_copy(m_v, mx_h.at[wid])


_edge_logits = pl.kernel(
    _edge_logits_body,
    out_type=(
        jax.ShapeDtypeStruct((ETP,), jnp.float32),
        jax.ShapeDtypeStruct((NW, NP), jnp.float32),
    ),
    mesh=_mesh,
    scratch_types=[
        pltpu.VMEM((CPT,), jnp.int32),
        pltpu.VMEM((CPT,), jnp.int32),
        pltpu.VMEM((NN,), jnp.float32),
        pltpu.VMEM((NN,), jnp.float32),
        pltpu.VMEM((CPT,), jnp.float32),
        pltpu.VMEM((NP,), jnp.float32),
    ],
    compiler_params=pltpu.CompilerParams(needs_layout_passes=False),
)


def _combine_body(mx_ref, m_ref):
    m = jnp.max(mx_ref[...], axis=0, keepdims=True)
    m_ref[...] = jnp.where(jnp.isfinite(m), m, 0.0)


def _combine(maxes):
    return pl.pallas_call(
        _combine_body,
        out_shape=jax.ShapeDtypeStruct((1, NP), jnp.float32),
    )(maxes)


def _aggregate_body_real(p_h, m_h, h_h, den_h, outp_h,
                         pA, pB, dA, dB, m_v, den_v, rowsA, rowsB, acc_sp,
                         gsA, gsB, ssA, ssB, stA, stB):
    cid = lax.axis_index("c")
    sid = lax.axis_index("s")
    wid = sid * 2 + cid
    pltpu.sync_copy(m_h, m_v)

    zero16 = jnp.zeros((16,), jnp.float32)

    def zeroden(i, carry):
        den_v[pl.ds(i * 16, 16)] = zero16
        return carry
    lax.fori_loop(0, NP // 16, zeroden, 0)

    def zerorows(i, carry):
        rowsA[i // 8, pl.ds((i % 8) * 16, 16)] = zero16
        return carry
    lax.fori_loop(0, CK * 8, zerorows, 0)

    def zeroacc(k, carry):
        pltpu.sync_copy(rowsA, acc_sp.at[pl.ds(sid * RPT + k * CK, CK)])
        return carry
    lax.fori_loop(0, RPT // CK, zeroacc, 0)
    if RPT % CK:
        pltpu.sync_copy(rowsA.at[pl.ds(0, RPT % CK)],
                        acc_sp.at[pl.ds(sid * RPT + (RPT // CK) * CK, RPT % CK)])

    plsc.subcore_barrier()

    HK = CK // 2

    def gather_cp(pX, rowsX, gsX):
        lo = pltpu.make_async_copy(
            h_h.at[pX.at[0, pl.ds(0, HK)]], rowsX.at[pl.ds(0, HK)], gsX)
        hi = pltpu.make_async_copy(
            h_h.at[pX.at[0, pl.ds(HK, HK)]], rowsX.at[pl.ds(HK, HK)], gsX)
        return lo, hi


    def stage_cp(g, pX, stX):
        return pltpu.make_async_copy(p_h.at[wid, g], pX, stX)

    def scatter_cp(dX, rowsX, ssX):
        return pltpu.async_copy(rowsX, acc_sp.at[dX], ssX, add=True)

    def scatter_wait(dX, rowsX, ssX):
        pltpu.make_async_copy(rowsX, acc_sp.at[dX], ssX).wait()

    def dcopy(pX, dX):
        for i in range(CK // 16):
            dX[pl.ds(i * 16, 16)] = pX[1, pl.ds(i * 16, 16)]

    def compute(pX, rowsX):
        def vec(v, carry):
            d16 = pX[1, pl.ds(v * 16, 16)]
            e16 = plsc.bitcast(pX[2, pl.ds(v * 16, 16)], jnp.float32)
            mm = plsc.load_gather(m_v, [d16])
            ex = jnp.exp(e16 - mm)
            plsc.addupdate_scatter(den_v, [d16], ex)
            gdn = lax.GatherDimensionNumbers(
                offset_dims=(), collapsed_slice_dims=(0,),
                start_index_map=(0,))
            for r in range(16):
                lane = lax.gather(
                    ex, jnp.full((16, 1), r, jnp.int32), gdn, (1,),
                    mode=lax.GatherScatterMode.PROMISE_IN_BOUNDS)
                row = v * 16 + r
                for c8 in range(8):
                    sl = pl.ds(c8 * 16, 16)
                    rowsX[row, sl] = rowsX[row, sl] * lane
            return carry
        lax.fori_loop(0, CK // 16, vec, 0)

    NH = NG // 2
    pltpu.sync_copy(p_h.at[wid, 0], pA)
    stage_cp(1, pB, stB).start()
    for _c in gather_cp(pA, rowsA, gsA):
        _c.start()

    def half(p, carry):
        for _c in gather_cp(pA, rowsA, gsA):
            _c.wait()

        @pl.when(p > 0)
        def _freeB():
            scatter_wait(dB, rowsB, ssB)
        stage_cp(2 * p + 1, pB, stB).wait()
        for _c in gather_cp(pB, rowsB, gsB):
            _c.start()
        compute(pA, rowsA)
        dcopy(pA, dA)
        scatter_cp(dA, rowsA, ssA)

        @pl.when(p < NH - 1)
        def _stA():
            stage_cp(2 * p + 2, pA, stA).start()

        for _c in gather_cp(pB, rowsB, gsB):
            _c.wait()
        scatter_wait(dA, rowsA, ssA)

        @pl.when(p < NH - 1)
        def _gA():
            stage_cp(2 * p + 2, pA, stA).wait()
            for _c in gather_cp(pA, rowsA, gsA):
                _c.start()
        compute(pB, rowsB)
        dcopy(pB, dB)
        scatter_cp(dB, rowsB, ssB)

        @pl.when(p < NH - 1)
        def _stB():
            stage_cp(2 * p + 3, pB, stB).start()
        return carry

    lax.fori_loop(0, NH, half, 0)
    scatter_wait(dB, rowsB, ssB)

    plsc.subcore_barrier()
    pltpu.sync_copy(acc_sp.at[pl.ds(sid * RPT, RPT)],
                    outp_h.at[cid, pl.ds(sid * RPT, RPT)])
    pltpu.sync_copy(den_v, den_h.at[wid])


_aggregate = pl.kernel(
    _aggregate_body_real,
    out_type=(
        jax.ShapeDtypeStruct((NW, NP), jnp.float32),
        jax.ShapeDtypeStruct((2, NP, DD), jnp.float32),
    ),
    mesh=_mesh,
    scratch_types=[
        pltpu.VMEM((3, CK), jnp.int32),
        pltpu.VMEM((3, CK), jnp.int32),
        pltpu.VMEM((CK,), jnp.int32),
        pltpu.VMEM((CK,), jnp.int32),
        pltpu.VMEM((NP,), jnp.float32),
        pltpu.VMEM((NP,), jnp.float32),
        pltpu.VMEM((CK, DD), jnp.float32),
        pltpu.VMEM((CK, DD), jnp.float32),
        pltpu.VMEM_SHARED((NP, DD), jnp.float32),
        pltpu.SemaphoreType.DMA,
        pltpu.SemaphoreType.DMA,
        pltpu.SemaphoreType.DMA,
        pltpu.SemaphoreType.DMA,
        pltpu.SemaphoreType.DMA,
        pltpu.SemaphoreType.DMA,
    ],
    compiler_params=pltpu.CompilerParams(needs_layout_passes=False),
)


def _final_body(outp_ref, den_ref, batch_ref, bias_ref, wl_ref, bl_ref,
                y_ref, pool, cnt):
    i = pl.program_id(0)

    @pl.when(i == 0)
    def _init():
        pool[...] = jnp.zeros((GG, DD), jnp.float32)
        cnt[...] = jnp.zeros((GG, DD), jnp.float32)

    ds = jnp.sum(den_ref[...], axis=0)
    acc = outp_ref[0] + outp_ref[1]
    rows = acc / (ds + 1e-16)[:, None] + bias_ref[...]
    b = batch_ref[0, 0, :]
    M = (lax.broadcasted_iota(jnp.int32, (GG, 1280), 0) == b[None, :]
         ).astype(jnp.float32)
    pool[...] += jnp.dot(M, rows, preferred_element_type=jnp.float32)
    cnt[...] += jnp.dot(M, jnp.ones((1280, DD), jnp.float32),
                        preferred_element_type=jnp.float32)

    @pl.when(i == NP // 1280 - 1)
    def _fin():
        pooled = pool[...] / jnp.maximum(cnt[...], 1.0)
        y = jnp.dot(pooled, wl_ref[...], preferred_element_type=jnp.float32)
        y_ref[...] = jnp.maximum(y + bl_ref[...], 0.0)


def _final(outp, denoms, batch3, bias, W_lin, b_lin):
    nb = NP // 1280
    return pl.pallas_call(
        _final_body,
        grid=(nb,),
        in_specs=[
            pl.BlockSpec((2, 1280, DD), lambda i: (0, i, 0)),
            pl.BlockSpec((NW, 1280), lambda i: (0, i)),
            pl.BlockSpec((1, 1, 1280), lambda i: (i, 0, 0)),
            pl.BlockSpec((1, DD), lambda i: (0, 0)),
            pl.BlockSpec((DD, NOUT), lambda i: (0, 0)),
            pl.BlockSpec((1, NOUT), lambda i: (0, 0)),
        ],
        out_specs=pl.BlockSpec((GG, NOUT), lambda i: (0, 0)),
        out_shape=jax.ShapeDtypeStruct((GG, NOUT), jnp.float32),
        scratch_shapes=[
            pltpu.VMEM((GG, DD), jnp.float32),
            pltpu.VMEM((GG, DD), jnp.float32),
        ],
    )(outp, denoms, batch3, bias[None, :], W_lin, b_lin[None, :])


def kernel(x, edge_index, batch, W, a_src, a_dst, bias, W_lin, b_lin):
    loop = jnp.arange(NN, dtype=jnp.int32)
    pad = jnp.zeros((ETP - ET,), jnp.int32)
    src = jnp.concatenate([edge_index[0], loop, pad])
    dst = jnp.concatenate([edge_index[1], loop, pad])
    batch_pad = jnp.pad(batch, (0, NP - NN), constant_values=GG)
    batch3 = batch_pad.reshape(NP // 1280, 1, 1280)

    h, s2, d2 = _prep(x, W, a_src, a_dst)

    e, maxes = _edge_logits(src, dst, s2[:, 0], d2[:, 0])
    m = _combine(maxes)[0]
    ebits = lax.bitcast_convert_type(e, jnp.int32)
    P = jnp.stack([src.reshape(NW, NG, CK), dst.reshape(NW, NG, CK),
                   ebits.reshape(NW, NG, CK)], axis=2)
    denoms, outp = _aggregate(P, m, h)
    return _final(outp, denoms, batch3, bias, W_lin, b_lin)
